# Initial kernel scaffold; baseline (speedup 1.0000x reference)
import jax
import jax.numpy as jnp
from jax.experimental import pallas as pl


def kernel(x, edge_index, W1, b1, W2, b2, Wc, bc):
    raise NotImplementedError("write your pallas kernel here")


# speedup vs baseline: 3.8908x; 3.8908x over previous
import functools

import jax
import jax.numpy as jnp
from jax import lax
from jax.experimental import pallas as pl
from jax.experimental.pallas import tpu as pltpu
from jax.experimental.pallas import tpu_sc as plsc

N_NODES = 10000
NP = 10240
E = 160000
RPP = 320
ACC = 328
DUMMY = 320
SCAN = 2000
STG = 320
GARB = 304
DEGACC = 352
LW = E + 128
CHUNK = 128


def _mesh():
    return plsc.VectorSubcoreMesh(core_axis_name="c", subcore_axis_name="s")


def _sc_route(src, dst, ones1):
    @functools.partial(
        pl.kernel,
        out_type=[
            jax.ShapeDtypeStruct((NP,), jnp.float32),
            jax.ShapeDtypeStruct((32 * LW,), jnp.int32),
            jax.ShapeDtypeStruct((32 * LW,), jnp.int32),
            jax.ShapeDtypeStruct((512,), jnp.int32),
        ],
        mesh=_mesh(),
        compiler_params=pltpu.CompilerParams(needs_layout_passes=False),
        scratch_types=[
            pltpu.VMEM((SCAN,), jnp.int32),
            pltpu.VMEM((SCAN,), jnp.int32),
            pltpu.VMEM((STG,), jnp.int32),
            pltpu.VMEM((STG,), jnp.int32),
            pltpu.VMEM((128,), jnp.int32),
            pltpu.VMEM((128,), jnp.int32),
            pltpu.VMEM((DEGACC,), jnp.float32),
            pltpu.VMEM((16,), jnp.int32),
        ],
    )
    def route(src_hbm, dst_hbm, ones_hbm, deg_hbm, srcs_hbm, dls_hbm, cnts_hbm,
              scan_s, scan_d, stg_s, stg_d, fl_s, fl_d, deg1, cnt_v):
        c = lax.axis_index("c")
        s = lax.axis_index("s")
        w = s * 2 + c
        lo = w * RPP

        pltpu.sync_copy(ones_hbm, deg1)
        onesv = jnp.ones((16,), jnp.float32)
        lane = lax.iota(jnp.int32, 16)

        def flush(total):
            for t in range(8):
                fl_s[pl.ds(16 * t, 16)] = stg_s[pl.ds(16 * t, 16)]
                fl_d[pl.ds(16 * t, 16)] = stg_d[pl.ds(16 * t, 16)]
            lbase = pl.multiple_of(w * LW + total, 8)
            pltpu.sync_copy(fl_s, srcs_hbm.at[pl.ds(lbase, 128)])
            pltpu.sync_copy(fl_d, dls_hbm.at[pl.ds(lbase, 128)])
            for e in range(128):
                dspl = plsc.load_gather(fl_d, [jnp.full((16,), e, jnp.int32)])
                didx = jnp.where(lane == 0, dspl, 336 + lane)
                plsc.addupdate_scatter(deg1, [didx], onesv)
            for t in range(8):
                stg_s[pl.ds(16 * t, 16)] = stg_s[pl.ds(128 + 16 * t, 16)]
                stg_d[pl.ds(16 * t, 16)] = stg_d[pl.ds(128 + 16 * t, 16)]

        def scan_body(k, carry):
            base = pl.multiple_of(k * SCAN, 8)
            pltpu.sync_copy(src_hbm.at[pl.ds(base, SCAN)], scan_s)
            pltpu.sync_copy(dst_hbm.at[pl.ds(base, SCAN)], scan_d)

            def grp(j, carry2):
                off, total = carry2
                d = scan_d[pl.ds(j * 16, 16)]
                sv = scan_s[pl.ds(j * 16, 16)]
                m = (d >= lo) & (d < lo + RPP)
                dl = jnp.where(m, d - lo, DUMMY)
                mi = m.astype(jnp.int32)
                ecum = plsc.cumsum(mi) - mi
                pos = jnp.where(m, off + ecum, GARB + lane)
                plsc.store_scatter(stg_s, [pos], sv)
                plsc.store_scatter(stg_d, [pos], dl)
                pc = plsc.all_reduce_population_count(m)
                off = off + lax.reduce_max(pc, (0,))
                do_fl = off >= 128

                @pl.when(do_fl)
                def _():
                    flush(total)

                off = jnp.where(do_fl, off - 128, off)
                total = total + jnp.where(do_fl, 128, 0)
                return (off, total)

            return lax.fori_loop(0, SCAN // 16, grp, carry, unroll=False)

        off, total = lax.fori_loop(0, E // SCAN, scan_body, (0, 0), unroll=False)

        for t in range(9):
            stg_s[pl.ds(off + 16 * t, 16)] = jnp.zeros((16,), jnp.int32)
            stg_d[pl.ds(off + 16 * t, 16)] = jnp.full((16,), DUMMY, jnp.int32)

        for _ in range(2):
            do_fl = off > 0

            @pl.when(do_fl)
            def _():
                flush(total)

            total = total + jnp.where(do_fl, 128, 0)
            off = jnp.maximum(off - 128, 0)

        cnt_v[pl.ds(0, 16)] = jnp.full((16,), total, jnp.int32)
        pltpu.sync_copy(cnt_v, cnts_hbm.at[pl.ds(pl.multiple_of(w * 16, 8), 16)])
        pltpu.sync_copy(deg1.at[pl.ds(0, RPP)], deg_hbm.at[pl.ds(w * RPP, RPP)])

    return route(src, dst, ones1)


def _sc_aggregate(g, srcs_list, dls_list, counts, zeros_acc):
    d_feat = g.shape[1]

    @functools.partial(
        pl.kernel,
        out_type=jax.ShapeDtypeStruct((NP, d_feat), jnp.float32),
        mesh=_mesh(),
        compiler_params=pltpu.CompilerParams(needs_layout_passes=False),
        scratch_types=[
            pltpu.VMEM((ACC, d_feat), jnp.float32),
            pltpu.VMEM((CHUNK,), jnp.int32),
            pltpu.VMEM((CHUNK,), jnp.int32),
            pltpu.VMEM((CHUNK, d_feat), jnp.float32),
            pltpu.VMEM((16,), jnp.int32),
            pltpu.SemaphoreType.DMA,
        ],
    )
    def agg(g_hbm, srcs_hbm, dls_hbm, cnts_hbm, z_hbm, out_hbm,
            acc, src_v, dl_v, rows_v, cnt_v, sem):
        c = lax.axis_index("c")
        s = lax.axis_index("s")
        w = s * 2 + c
        pltpu.sync_copy(z_hbm, acc)
        pltpu.sync_copy(cnts_hbm.at[pl.ds(pl.multiple_of(w * 16, 8), 16)], cnt_v)
        n = lax.reduce_max(cnt_v[...], (0,))
        cols = [lax.iota(jnp.int32, 16) + 16 * j for j in range(d_feat // 16)]

        def body(k, carry):
            off = pl.multiple_of(w * LW + k * CHUNK, 8)
            pltpu.sync_copy(srcs_hbm.at[pl.ds(off, CHUNK)], src_v)
            pltpu.sync_copy(dls_hbm.at[pl.ds(off, CHUNK)], dl_v)
            pltpu.async_copy(g_hbm.at[src_v], rows_v, sem).wait()

            def egrp(q, c2):
                for t in range(16):
                    e = q * 16 + t
                    dspl = plsc.load_gather(dl_v, [jnp.full((16,), e, jnp.int32)])
                    for j in range(d_feat // 16):
                        v = rows_v[e, pl.ds(16 * j, 16)]
                        plsc.addupdate_scatter(acc, [dspl, cols[j]], v)
                return c2

            lax.fori_loop(0, CHUNK // 16, egrp, 0, unroll=False)
            return carry

        lax.fori_loop(0, n // CHUNK, body, 0, unroll=False)
        pltpu.sync_copy(acc.at[pl.ds(0, RPP)], out_hbm.at[pl.ds(w * RPP, RPP)])

    return agg(g, srcs_list, dls_list, counts, zeros_acc)


_BN = 1024


def _mm_scale_body(x_ref, w_ref, deg_ref, o_ref):
    h = jnp.dot(x_ref[...], w_ref[...], preferred_element_type=jnp.float32)
    dinv = lax.rsqrt(deg_ref[...]).reshape(-1, 1)
    o_ref[...] = h * dinv


def _tc_mm_scale(x, w, deg):
    n, d = x.shape
    h = w.shape[1]
    return pl.pallas_call(
        _mm_scale_body,
        grid=(n // _BN,),
        in_specs=[
            pl.BlockSpec((_BN, d), lambda i: (i, 0)),
            pl.BlockSpec((d, h), lambda i: (0, 0)),
            pl.BlockSpec((_BN,), lambda i: (i,)),
        ],
        out_specs=pl.BlockSpec((_BN, h), lambda i: (i, 0)),
        out_shape=jax.ShapeDtypeStruct((n, h), jnp.float32),
    )(x, w, deg)


def _combine_mm_body(acc_ref, g_ref, deg_ref, b_ref, w_ref, o_ref):
    dinv = lax.rsqrt(deg_ref[...]).reshape(-1, 1)
    a = jnp.maximum(dinv * (acc_ref[...] + g_ref[...]) + b_ref[...], 0.0)
    o_ref[...] = dinv * jnp.dot(a, w_ref[...], preferred_element_type=jnp.float32)


def _tc_combine_mm(acc, g, deg, b, w):
    n, d = g.shape
    h = w.shape[1]
    return pl.pallas_call(
        _combine_mm_body,
        grid=(n // _BN,),
        in_specs=[
            pl.BlockSpec((_BN, d), lambda i: (i, 0)),
            pl.BlockSpec((_BN, d), lambda i: (i, 0)),
            pl.BlockSpec((_BN,), lambda i: (i,)),
            pl.BlockSpec((1, d), lambda i: (0, 0)),
            pl.BlockSpec((d, h), lambda i: (0, 0)),
        ],
        out_specs=pl.BlockSpec((_BN, h), lambda i: (i, 0)),
        out_shape=jax.ShapeDtypeStruct((n, h), jnp.float32),
    )(acc, g, deg, b, w)


def _final_body(acc_ref, g_ref, deg_ref, b_ref, wc_ref, bc_ref, emb_ref, log_ref):
    dinv = lax.rsqrt(deg_ref[...]).reshape(-1, 1)
    emb = jnp.maximum(dinv * (acc_ref[...] + g_ref[...]) + b_ref[...], 0.0)
    emb_ref[...] = emb
    log_ref[...] = jnp.dot(emb, wc_ref[...], preferred_element_type=jnp.float32) + bc_ref[...]


def _tc_final(acc, g, deg, b, wc, bc):
    n, d = g.shape
    ncls = wc.shape[1]
    return pl.pallas_call(
        _final_body,
        grid=(n // _BN,),
        in_specs=[
            pl.BlockSpec((_BN, d), lambda i: (i, 0)),
            pl.BlockSpec((_BN, d), lambda i: (i, 0)),
            pl.BlockSpec((_BN,), lambda i: (i,)),
            pl.BlockSpec((1, d), lambda i: (0, 0)),
            pl.BlockSpec((d, ncls), lambda i: (0, 0)),
            pl.BlockSpec((1, ncls), lambda i: (0, 0)),
        ],
        out_specs=[
            pl.BlockSpec((_BN, d), lambda i: (i, 0)),
            pl.BlockSpec((_BN, ncls), lambda i: (i, 0)),
        ],
        out_shape=[
            jax.ShapeDtypeStruct((n, d), jnp.float32),
            jax.ShapeDtypeStruct((n, ncls), jnp.float32),
        ],
    )(acc, g, deg, b, wc, bc)


def kernel(x, edge_index, W1, b1, W2, b2, Wc, bc):
    src = edge_index[0].astype(jnp.int32)
    dst = edge_index[1].astype(jnp.int32)

    x_pad = jnp.pad(x, ((0, NP - N_NODES), (0, 0)))
    ones1 = jnp.ones((DEGACC,), jnp.float32)
    zeros_acc = jnp.zeros((ACC, x.shape[1]), jnp.float32)
    b1r = b1.reshape(1, -1)
    b2r = b2.reshape(1, -1)
    bcr = bc.reshape(1, -1)

    deg, srcs_list, dls_list, counts = _sc_route(src, dst, ones1)
    g1 = _tc_mm_scale(x_pad, W1, deg)
    acc1 = _sc_aggregate(g1, srcs_list, dls_list, counts, zeros_acc)
    g2 = _tc_combine_mm(acc1, g1, deg, b1r, W2)
    acc2 = _sc_aggregate(g2, srcs_list, dls_list, counts, zeros_acc)
    emb_p, logits_p = _tc_final(acc2, g2, deg, b2r, Wc, bcr)
    return (logits_p[:N_NODES], emb_p[:N_NODES])


# speedup vs baseline: 4.4178x; 1.1355x over previous
import functools

import jax
import jax.numpy as jnp
from jax import lax
from jax.experimental import pallas as pl
from jax.experimental.pallas import tpu as pltpu
from jax.experimental.pallas import tpu_sc as plsc

N_NODES = 10000
NP = 10240
E = 160000
RPP = 320
ACC = 328
DUMMY = 320
SCAN = 2000
STG = 320
GARB = 304
DEGACC = 352
LW = E + 128
CHUNK = 128


def _mesh():
    return plsc.VectorSubcoreMesh(core_axis_name="c", subcore_axis_name="s")


def _sc_route(src, dst, ones1):
    @functools.partial(
        pl.kernel,
        out_type=[
            jax.ShapeDtypeStruct((NP,), jnp.float32),
            jax.ShapeDtypeStruct((32 * LW,), jnp.int32),
            jax.ShapeDtypeStruct((32 * LW,), jnp.int32),
            jax.ShapeDtypeStruct((512,), jnp.int32),
        ],
        mesh=_mesh(),
        compiler_params=pltpu.CompilerParams(needs_layout_passes=False),
        scratch_types=[
            pltpu.VMEM((SCAN,), jnp.int32),
            pltpu.VMEM((SCAN,), jnp.int32),
            pltpu.VMEM((STG,), jnp.int32),
            pltpu.VMEM((STG,), jnp.int32),
            pltpu.VMEM((128,), jnp.int32),
            pltpu.VMEM((128,), jnp.int32),
            pltpu.VMEM((DEGACC,), jnp.float32),
            pltpu.VMEM((16,), jnp.int32),
        ],
    )
    def route(src_hbm, dst_hbm, ones_hbm, deg_hbm, srcs_hbm, dls_hbm, cnts_hbm,
              scan_s, scan_d, stg_s, stg_d, fl_s, fl_d, deg1, cnt_v):
        c = lax.axis_index("c")
        s = lax.axis_index("s")
        w = s * 2 + c
        lo = w * RPP

        pltpu.sync_copy(ones_hbm, deg1)
        onesv = jnp.ones((16,), jnp.float32)
        lane = lax.iota(jnp.int32, 16)

        def flush(total):
            for t in range(8):
                fl_s[pl.ds(16 * t, 16)] = stg_s[pl.ds(16 * t, 16)]
                fl_d[pl.ds(16 * t, 16)] = stg_d[pl.ds(16 * t, 16)]
            lbase = pl.multiple_of(w * LW + total, 8)
            pltpu.sync_copy(fl_s, srcs_hbm.at[pl.ds(lbase, 128)])
            pltpu.sync_copy(fl_d, dls_hbm.at[pl.ds(lbase, 128)])
            for e in range(128):
                dspl = plsc.load_gather(fl_d, [jnp.full((16,), e, jnp.int32)])
                didx = jnp.where(lane == 0, dspl, 336 + lane)
                plsc.addupdate_scatter(deg1, [didx], onesv)
            for t in range(8):
                stg_s[pl.ds(16 * t, 16)] = stg_s[pl.ds(128 + 16 * t, 16)]
                stg_d[pl.ds(16 * t, 16)] = stg_d[pl.ds(128 + 16 * t, 16)]

        def scan_body(k, carry):
            base = pl.multiple_of(k * SCAN, 8)
            pltpu.sync_copy(src_hbm.at[pl.ds(base, SCAN)], scan_s)
            pltpu.sync_copy(dst_hbm.at[pl.ds(base, SCAN)], scan_d)

            def grp(j, carry2):
                off, total = carry2
                d = scan_d[pl.ds(j * 16, 16)]
                sv = scan_s[pl.ds(j * 16, 16)]
                m = (d >= lo) & (d < lo + RPP)
                dl = jnp.where(m, d - lo, DUMMY)
                mi = m.astype(jnp.int32)
                ecum = plsc.cumsum(mi) - mi
                pos = jnp.where(m, off + ecum, GARB + lane)
                plsc.store_scatter(stg_s, [pos], sv)
                plsc.store_scatter(stg_d, [pos], dl)
                pc = plsc.all_reduce_population_count(m)
                off = off + lax.reduce_max(pc, (0,))
                do_fl = off >= 128

                @pl.when(do_fl)
                def _():
                    flush(total)

                off = jnp.where(do_fl, off - 128, off)
                total = total + jnp.where(do_fl, 128, 0)
                return (off, total)

            return lax.fori_loop(0, SCAN // 16, grp, carry, unroll=False)

        off, total = lax.fori_loop(0, E // SCAN, scan_body, (0, 0), unroll=False)

        for t in range(9):
            stg_s[pl.ds(off + 16 * t, 16)] = jnp.zeros((16,), jnp.int32)
            stg_d[pl.ds(off + 16 * t, 16)] = jnp.full((16,), DUMMY, jnp.int32)

        for _ in range(2):
            do_fl = off > 0

            @pl.when(do_fl)
            def _():
                flush(total)

            total = total + jnp.where(do_fl, 128, 0)
            off = jnp.maximum(off - 128, 0)

        cnt_v[pl.ds(0, 16)] = jnp.full((16,), total, jnp.int32)
        pltpu.sync_copy(cnt_v, cnts_hbm.at[pl.ds(pl.multiple_of(w * 16, 8), 16)])
        pltpu.sync_copy(deg1.at[pl.ds(0, RPP)], deg_hbm.at[pl.ds(w * RPP, RPP)])

    return route(src, dst, ones1)


def _sc_aggregate(g, srcs_list, dls_list, counts, zeros_acc):
    d_feat = g.shape[1]
    H = CHUNK // 2

    @functools.partial(
        pl.kernel,
        out_type=jax.ShapeDtypeStruct((NP, d_feat), jnp.float32),
        mesh=_mesh(),
        compiler_params=pltpu.CompilerParams(needs_layout_passes=False),
        scratch_types=[
            pltpu.VMEM((ACC, d_feat), jnp.float32),
            pltpu.VMEM((2, CHUNK), jnp.int32),
            pltpu.VMEM((2, CHUNK), jnp.int32),
            pltpu.VMEM((CHUNK, d_feat), jnp.float32),
            pltpu.VMEM((16,), jnp.int32),
            pltpu.SemaphoreType.DMA,
            pltpu.SemaphoreType.DMA,
        ],
    )
    def agg(g_hbm, srcs_hbm, dls_hbm, cnts_hbm, z_hbm, out_hbm,
            acc, src_v, dl_v, rows_v, cnt_v, sem_g, sem_i):
        c = lax.axis_index("c")
        s = lax.axis_index("s")
        w = s * 2 + c
        pltpu.sync_copy(z_hbm, acc)
        pltpu.sync_copy(cnts_hbm.at[pl.ds(pl.multiple_of(w * 16, 8), 16)], cnt_v)
        n = lax.reduce_max(cnt_v[...], (0,))
        nc = n // CHUNK
        cols = [lax.iota(jnp.int32, 16) + 16 * j for j in range(d_feat // 16)]

        def adds(b, half_base):
            def egrp(q, c2):
                for t in range(16):
                    e = half_base + q * 16 + t
                    dspl = plsc.load_gather(dl_v.at[b], [jnp.full((16,), e, jnp.int32)])
                    for j in range(d_feat // 16):
                        v = rows_v[e, pl.ds(16 * j, 16)]
                        plsc.addupdate_scatter(acc, [dspl, cols[j]], v)
                return c2

            lax.fori_loop(0, H // 16, egrp, 0, unroll=False)

        lbase0 = pl.multiple_of(w * LW, 8)

        @pl.when(nc > 0)
        def _():
            pltpu.sync_copy(srcs_hbm.at[pl.ds(lbase0, CHUNK)], src_v.at[0])
            pltpu.sync_copy(dls_hbm.at[pl.ds(lbase0, CHUNK)], dl_v.at[0])
            pltpu.async_copy(g_hbm.at[src_v.at[0, pl.ds(0, H)]],
                             rows_v.at[pl.ds(0, H)], sem_g)

        def body(k, carry):
            b = lax.rem(k, 2)
            nb = 1 - b
            knext = jnp.minimum(k + 1, nc - 1)
            lnext = pl.multiple_of(w * LW, 8) + knext * CHUNK
            ci = pltpu.async_copy(srcs_hbm.at[pl.ds(lnext, CHUNK)], src_v.at[nb], sem_i)
            di = pltpu.async_copy(dls_hbm.at[pl.ds(lnext, CHUNK)], dl_v.at[nb], sem_i)
            pltpu.make_async_copy(g_hbm.at[src_v.at[b, pl.ds(0, H)]],
                                  rows_v.at[pl.ds(0, H)], sem_g).wait()
            pltpu.async_copy(g_hbm.at[src_v.at[b, pl.ds(H, H)]],
                             rows_v.at[pl.ds(H, H)], sem_g)
            adds(b, 0)
            pltpu.make_async_copy(g_hbm.at[src_v.at[b, pl.ds(H, H)]],
                                  rows_v.at[pl.ds(H, H)], sem_g).wait()
            ci.wait()
            di.wait()
            pltpu.async_copy(g_hbm.at[src_v.at[nb, pl.ds(0, H)]],
                             rows_v.at[pl.ds(0, H)], sem_g)
            adds(b, H)
            return carry

        lax.fori_loop(0, nc, body, 0, unroll=False)

        @pl.when(nc > 0)
        def _():
            pltpu.make_async_copy(g_hbm.at[src_v.at[0, pl.ds(0, H)]],
                                  rows_v.at[pl.ds(0, H)], sem_g).wait()

        pltpu.sync_copy(acc.at[pl.ds(0, RPP)], out_hbm.at[pl.ds(w * RPP, RPP)])

    return agg(g, srcs_list, dls_list, counts, zeros_acc)


_BN = 1024


def _mm_scale_body(x_ref, w_ref, deg_ref, o_ref):
    h = jnp.dot(x_ref[...], w_ref[...], preferred_element_type=jnp.float32)
    dinv = lax.rsqrt(deg_ref[...]).reshape(-1, 1)
    o_ref[...] = h * dinv


def _tc_mm_scale(x, w, deg):
    n, d = x.shape
    h = w.shape[1]
    return pl.pallas_call(
        _mm_scale_body,
        grid=(n // _BN,),
        in_specs=[
            pl.BlockSpec((_BN, d), lambda i: (i, 0)),
            pl.BlockSpec((d, h), lambda i: (0, 0)),
            pl.BlockSpec((_BN,), lambda i: (i,)),
        ],
        out_specs=pl.BlockSpec((_BN, h), lambda i: (i, 0)),
        out_shape=jax.ShapeDtypeStruct((n, h), jnp.float32),
    )(x, w, deg)


def _combine_mm_body(acc_ref, g_ref, deg_ref, b_ref, w_ref, o_ref):
    dinv = lax.rsqrt(deg_ref[...]).reshape(-1, 1)
    a = jnp.maximum(dinv * (acc_ref[...] + g_ref[...]) + b_ref[...], 0.0)
    o_ref[...] = dinv * jnp.dot(a, w_ref[...], preferred_element_type=jnp.float32)


def _tc_combine_mm(acc, g, deg, b, w):
    n, d = g.shape
    h = w.shape[1]
    return pl.pallas_call(
        _combine_mm_body,
        grid=(n // _BN,),
        in_specs=[
            pl.BlockSpec((_BN, d), lambda i: (i, 0)),
            pl.BlockSpec((_BN, d), lambda i: (i, 0)),
            pl.BlockSpec((_BN,), lambda i: (i,)),
            pl.BlockSpec((1, d), lambda i: (0, 0)),
            pl.BlockSpec((d, h), lambda i: (0, 0)),
        ],
        out_specs=pl.BlockSpec((_BN, h), lambda i: (i, 0)),
        out_shape=jax.ShapeDtypeStruct((n, h), jnp.float32),
    )(acc, g, deg, b, w)


def _final_body(acc_ref, g_ref, deg_ref, b_ref, wc_ref, bc_ref, emb_ref, log_ref):
    dinv = lax.rsqrt(deg_ref[...]).reshape(-1, 1)
    emb = jnp.maximum(dinv * (acc_ref[...] + g_ref[...]) + b_ref[...], 0.0)
    emb_ref[...] = emb
    log_ref[...] = jnp.dot(emb, wc_ref[...], preferred_element_type=jnp.float32) + bc_ref[...]


def _tc_final(acc, g, deg, b, wc, bc):
    n, d = g.shape
    ncls = wc.shape[1]
    return pl.pallas_call(
        _final_body,
        grid=(n // _BN,),
        in_specs=[
            pl.BlockSpec((_BN, d), lambda i: (i, 0)),
            pl.BlockSpec((_BN, d), lambda i: (i, 0)),
            pl.BlockSpec((_BN,), lambda i: (i,)),
            pl.BlockSpec((1, d), lambda i: (0, 0)),
            pl.BlockSpec((d, ncls), lambda i: (0, 0)),
            pl.BlockSpec((1, ncls), lambda i: (0, 0)),
        ],
        out_specs=[
            pl.BlockSpec((_BN, d), lambda i: (i, 0)),
            pl.BlockSpec((_BN, ncls), lambda i: (i, 0)),
        ],
        out_shape=[
            jax.ShapeDtypeStruct((n, d), jnp.float32),
            jax.ShapeDtypeStruct((n, ncls), jnp.float32),
        ],
    )(acc, g, deg, b, wc, bc)


def kernel(x, edge_index, W1, b1, W2, b2, Wc, bc):
    src = edge_index[0].astype(jnp.int32)
    dst = edge_index[1].astype(jnp.int32)

    x_pad = jnp.pad(x, ((0, NP - N_NODES), (0, 0)))
    ones1 = jnp.ones((DEGACC,), jnp.float32)
    zeros_acc = jnp.zeros((ACC, x.shape[1]), jnp.float32)
    b1r = b1.reshape(1, -1)
    b2r = b2.reshape(1, -1)
    bcr = bc.reshape(1, -1)

    deg, srcs_list, dls_list, counts = _sc_route(src, dst, ones1)
    g1 = _tc_mm_scale(x_pad, W1, deg)
    acc1 = _sc_aggregate(g1, srcs_list, dls_list, counts, zeros_acc)
    g2 = _tc_combine_mm(acc1, g1, deg, b1r, W2)
    acc2 = _sc_aggregate(g2, srcs_list, dls_list, counts, zeros_acc)
    emb_p, logits_p = _tc_final(acc2, g2, deg, b2r, Wc, bcr)
    return (logits_p[:N_NODES], emb_p[:N_NODES])


# speedup vs baseline: 4.4736x; 1.0126x over previous
import functools

import jax
import jax.numpy as jnp
from jax import lax
from jax.experimental import pallas as pl
from jax.experimental.pallas import tpu as pltpu
from jax.experimental.pallas import tpu_sc as plsc

N_NODES = 10000
NP = 10240
E = 160000
RPP = 320
ACC = 328
DUMMY = 320
SCAN = 2000
STG = 320
GARB = 304
DEGACC = 352
LW = E + 128
CHUNK = 128


def _mesh():
    return plsc.VectorSubcoreMesh(core_axis_name="c", subcore_axis_name="s")


def _sc_route(src, dst, ones1):
    @functools.partial(
        pl.kernel,
        out_type=[
            jax.ShapeDtypeStruct((NP,), jnp.float32),
            jax.ShapeDtypeStruct((32 * LW,), jnp.int32),
            jax.ShapeDtypeStruct((32 * LW,), jnp.int32),
            jax.ShapeDtypeStruct((512,), jnp.int32),
        ],
        mesh=_mesh(),
        compiler_params=pltpu.CompilerParams(needs_layout_passes=False),
        scratch_types=[
            pltpu.VMEM((SCAN,), jnp.int32),
            pltpu.VMEM((SCAN,), jnp.int32),
            pltpu.VMEM((STG,), jnp.int32),
            pltpu.VMEM((STG,), jnp.int32),
            pltpu.VMEM((128,), jnp.int32),
            pltpu.VMEM((128,), jnp.int32),
            pltpu.VMEM((DEGACC,), jnp.float32),
            pltpu.VMEM((16,), jnp.int32),
        ],
    )
    def route(src_hbm, dst_hbm, ones_hbm, deg_hbm, srcs_hbm, dls_hbm, cnts_hbm,
              scan_s, scan_d, stg_s, stg_d, fl_s, fl_d, deg1, cnt_v):
        c = lax.axis_index("c")
        s = lax.axis_index("s")
        w = s * 2 + c
        lo = w * RPP

        pltpu.sync_copy(ones_hbm, deg1)
        onesv = jnp.ones((16,), jnp.float32)
        lane = lax.iota(jnp.int32, 16)

        def flush(total):
            for t in range(8):
                fl_s[pl.ds(16 * t, 16)] = stg_s[pl.ds(16 * t, 16)]
                fl_d[pl.ds(16 * t, 16)] = stg_d[pl.ds(16 * t, 16)]
            lbase = pl.multiple_of(w * LW + total, 8)
            pltpu.sync_copy(fl_s, srcs_hbm.at[pl.ds(lbase, 128)])
            pltpu.sync_copy(fl_d, dls_hbm.at[pl.ds(lbase, 128)])
            for e in range(128):
                dspl = plsc.load_gather(fl_d, [jnp.full((16,), e, jnp.int32)])
                didx = jnp.where(lane == 0, dspl, 336 + lane)
                plsc.addupdate_scatter(deg1, [didx], onesv)
            for t in range(8):
                stg_s[pl.ds(16 * t, 16)] = stg_s[pl.ds(128 + 16 * t, 16)]
                stg_d[pl.ds(16 * t, 16)] = stg_d[pl.ds(128 + 16 * t, 16)]

        def scan_body(k, carry):
            base = pl.multiple_of(k * SCAN, 8)
            pltpu.sync_copy(src_hbm.at[pl.ds(base, SCAN)], scan_s)
            pltpu.sync_copy(dst_hbm.at[pl.ds(base, SCAN)], scan_d)

            def grp(j, carry2):
                off, total = carry2
                d = scan_d[pl.ds(j * 16, 16)]
                sv = scan_s[pl.ds(j * 16, 16)]
                m = (d >= lo) & (d < lo + RPP)
                dl = jnp.where(m, d - lo, DUMMY)
                mi = m.astype(jnp.int32)
                ecum = plsc.cumsum(mi) - mi
                pos = jnp.where(m, off + ecum, GARB + lane)
                plsc.store_scatter(stg_s, [pos], sv)
                plsc.store_scatter(stg_d, [pos], dl)
                pc = plsc.all_reduce_population_count(m)
                off = off + lax.reduce_max(pc, (0,))
                do_fl = off >= 128

                @pl.when(do_fl)
                def _():
                    flush(total)

                off = jnp.where(do_fl, off - 128, off)
                total = total + jnp.where(do_fl, 128, 0)
                return (off, total)

            return lax.fori_loop(0, SCAN // 16, grp, carry, unroll=False)

        off, total = lax.fori_loop(0, E // SCAN, scan_body, (0, 0), unroll=False)

        for t in range(9):
            stg_s[pl.ds(off + 16 * t, 16)] = jnp.zeros((16,), jnp.int32)
            stg_d[pl.ds(off + 16 * t, 16)] = jnp.full((16,), DUMMY, jnp.int32)

        for _ in range(2):
            do_fl = off > 0

            @pl.when(do_fl)
            def _():
                flush(total)

            total = total + jnp.where(do_fl, 128, 0)
            off = jnp.maximum(off - 128, 0)

        cnt_v[pl.ds(0, 16)] = jnp.full((16,), total, jnp.int32)
        pltpu.sync_copy(cnt_v, cnts_hbm.at[pl.ds(pl.multiple_of(w * 16, 8), 16)])
        pltpu.sync_copy(deg1.at[pl.ds(0, RPP)], deg_hbm.at[pl.ds(w * RPP, RPP)])

    return route(src, dst, ones1)


def _sc_aggregate(g, srcs_list, dls_list, counts, zeros_acc):
    d_feat = g.shape[1]
    H = CHUNK // 2

    @functools.partial(
        pl.kernel,
        out_type=jax.ShapeDtypeStruct((NP * d_feat,), jnp.float32),
        mesh=_mesh(),
        compiler_params=pltpu.CompilerParams(needs_layout_passes=False),
        scratch_types=[
            pltpu.VMEM((ACC * d_feat,), jnp.float32),
            pltpu.VMEM((2, CHUNK), jnp.int32),
            pltpu.VMEM((2, CHUNK), jnp.int32),
            pltpu.VMEM((2, CHUNK), jnp.int32),
            pltpu.VMEM((CHUNK, d_feat), jnp.float32),
            pltpu.VMEM((16,), jnp.int32),
            pltpu.SemaphoreType.DMA,
            pltpu.SemaphoreType.DMA,
        ],
    )
    def agg(g_hbm, srcs_hbm, dls_hbm, cnts_hbm, z_hbm, out_hbm,
            acc, src_v, dl_v, fb_v, rows_v, cnt_v, sem_g, sem_i):
        c = lax.axis_index("c")
        s = lax.axis_index("s")
        w = s * 2 + c
        pltpu.sync_copy(z_hbm, acc)
        pltpu.sync_copy(cnts_hbm.at[pl.ds(pl.multiple_of(w * 16, 8), 16)], cnt_v)
        n = lax.reduce_max(cnt_v[...], (0,))
        nc = n // CHUNK
        cols = [lax.iota(jnp.int32, 16) + 16 * j for j in range(d_feat // 16)]

        def adds(b, half_base):
            def egrp(q, c2):
                base = half_base + q * 16
                spl = [plsc.load_gather(fb_v.at[b], [jnp.full((16,), base + t, jnp.int32)])
                       for t in range(16)]
                for t in range(16):
                    for j in range(d_feat // 16):
                        v = rows_v[base + t, pl.ds(16 * j, 16)]
                        plsc.addupdate_scatter(acc, [spl[t] + cols[j]], v)
                return c2

            lax.fori_loop(0, H // 16, egrp, 0, unroll=False)

        def fbcompute(b):
            for q in range(CHUNK // 16):
                fb_v[b, pl.ds(16 * q, 16)] = dl_v[b, pl.ds(16 * q, 16)] * d_feat

        lbase0 = pl.multiple_of(w * LW, 8)

        @pl.when(nc > 0)
        def _():
            pltpu.sync_copy(srcs_hbm.at[pl.ds(lbase0, CHUNK)], src_v.at[0])
            pltpu.sync_copy(dls_hbm.at[pl.ds(lbase0, CHUNK)], dl_v.at[0])
            pltpu.async_copy(g_hbm.at[src_v.at[0, pl.ds(0, H)]],
                             rows_v.at[pl.ds(0, H)], sem_g)

        def body(k, carry):
            b = lax.rem(k, 2)
            nb = 1 - b
            fbcompute(b)
            knext = jnp.minimum(k + 1, nc - 1)
            lnext = pl.multiple_of(w * LW, 8) + knext * CHUNK
            ci = pltpu.async_copy(srcs_hbm.at[pl.ds(lnext, CHUNK)], src_v.at[nb], sem_i)
            di = pltpu.async_copy(dls_hbm.at[pl.ds(lnext, CHUNK)], dl_v.at[nb], sem_i)
            pltpu.make_async_copy(g_hbm.at[src_v.at[b, pl.ds(0, H)]],
                                  rows_v.at[pl.ds(0, H)], sem_g).wait()
            pltpu.async_copy(g_hbm.at[src_v.at[b, pl.ds(H, H)]],
                             rows_v.at[pl.ds(H, H)], sem_g)
            adds(b, 0)
            pltpu.make_async_copy(g_hbm.at[src_v.at[b, pl.ds(H, H)]],
                                  rows_v.at[pl.ds(H, H)], sem_g).wait()
            ci.wait()
            di.wait()
            pltpu.async_copy(g_hbm.at[src_v.at[nb, pl.ds(0, H)]],
                             rows_v.at[pl.ds(0, H)], sem_g)
            adds(b, H)
            return carry

        lax.fori_loop(0, nc, body, 0, unroll=False)

        @pl.when(nc > 0)
        def _():
            pltpu.make_async_copy(g_hbm.at[src_v.at[0, pl.ds(0, H)]],
                                  rows_v.at[pl.ds(0, H)], sem_g).wait()

        pltpu.sync_copy(acc.at[pl.ds(0, RPP * d_feat)],
                        out_hbm.at[pl.ds(w * RPP * d_feat, RPP * d_feat)])

    return agg(g, srcs_list, dls_list, counts, zeros_acc)


_BN = 1024


def _mm_scale_body(x_ref, w_ref, deg_ref, o_ref):
    h = jnp.dot(x_ref[...], w_ref[...], preferred_element_type=jnp.float32)
    dinv = lax.rsqrt(deg_ref[...]).reshape(-1, 1)
    o_ref[...] = h * dinv


def _tc_mm_scale(x, w, deg):
    n, d = x.shape
    h = w.shape[1]
    return pl.pallas_call(
        _mm_scale_body,
        grid=(n // _BN,),
        in_specs=[
            pl.BlockSpec((_BN, d), lambda i: (i, 0)),
            pl.BlockSpec((d, h), lambda i: (0, 0)),
            pl.BlockSpec((_BN,), lambda i: (i,)),
        ],
        out_specs=pl.BlockSpec((_BN, h), lambda i: (i, 0)),
        out_shape=jax.ShapeDtypeStruct((n, h), jnp.float32),
    )(x, w, deg)


def _combine_mm_body(acc_ref, g_ref, deg_ref, b_ref, w_ref, o_ref):
    dinv = lax.rsqrt(deg_ref[...]).reshape(-1, 1)
    a = jnp.maximum(dinv * (acc_ref[...] + g_ref[...]) + b_ref[...], 0.0)
    o_ref[...] = dinv * jnp.dot(a, w_ref[...], preferred_element_type=jnp.float32)


def _tc_combine_mm(acc, g, deg, b, w):
    n, d = g.shape
    h = w.shape[1]
    return pl.pallas_call(
        _combine_mm_body,
        grid=(n // _BN,),
        in_specs=[
            pl.BlockSpec((_BN, d), lambda i: (i, 0)),
            pl.BlockSpec((_BN, d), lambda i: (i, 0)),
            pl.BlockSpec((_BN,), lambda i: (i,)),
            pl.BlockSpec((1, d), lambda i: (0, 0)),
            pl.BlockSpec((d, h), lambda i: (0, 0)),
        ],
        out_specs=pl.BlockSpec((_BN, h), lambda i: (i, 0)),
        out_shape=jax.ShapeDtypeStruct((n, h), jnp.float32),
    )(acc, g, deg, b, w)


def _final_body(acc_ref, g_ref, deg_ref, b_ref, wc_ref, bc_ref, emb_ref, log_ref):
    dinv = lax.rsqrt(deg_ref[...]).reshape(-1, 1)
    emb = jnp.maximum(dinv * (acc_ref[...] + g_ref[...]) + b_ref[...], 0.0)
    emb_ref[...] = emb
    log_ref[...] = jnp.dot(emb, wc_ref[...], preferred_element_type=jnp.float32) + bc_ref[...]


def _tc_final(acc, g, deg, b, wc, bc):
    n, d = g.shape
    ncls = wc.shape[1]
    return pl.pallas_call(
        _final_body,
        grid=(n // _BN,),
        in_specs=[
            pl.BlockSpec((_BN, d), lambda i: (i, 0)),
            pl.BlockSpec((_BN, d), lambda i: (i, 0)),
            pl.BlockSpec((_BN,), lambda i: (i,)),
            pl.BlockSpec((1, d), lambda i: (0, 0)),
            pl.BlockSpec((d, ncls), lambda i: (0, 0)),
            pl.BlockSpec((1, ncls), lambda i: (0, 0)),
        ],
        out_specs=[
            pl.BlockSpec((_BN, d), lambda i: (i, 0)),
            pl.BlockSpec((_BN, ncls), lambda i: (i, 0)),
        ],
        out_shape=[
            jax.ShapeDtypeStruct((n, d), jnp.float32),
            jax.ShapeDtypeStruct((n, ncls), jnp.float32),
        ],
    )(acc, g, deg, b, wc, bc)


def kernel(x, edge_index, W1, b1, W2, b2, Wc, bc):
    src = edge_index[0].astype(jnp.int32)
    dst = edge_index[1].astype(jnp.int32)

    x_pad = jnp.pad(x, ((0, NP - N_NODES), (0, 0)))
    ones1 = jnp.ones((DEGACC,), jnp.float32)
    zeros_acc = jnp.zeros((ACC * x.shape[1],), jnp.float32)
    b1r = b1.reshape(1, -1)
    b2r = b2.reshape(1, -1)
    bcr = bc.reshape(1, -1)

    deg, srcs_list, dls_list, counts = _sc_route(src, dst, ones1)
    g1 = _tc_mm_scale(x_pad, W1, deg)
    acc1 = _sc_aggregate(g1, srcs_list, dls_list, counts, zeros_acc).reshape(NP, -1)
    g2 = _tc_combine_mm(acc1, g1, deg, b1r, W2)
    acc2 = _sc_aggregate(g2, srcs_list, dls_list, counts, zeros_acc).reshape(NP, -1)
    emb_p, logits_p = _tc_final(acc2, g2, deg, b2r, Wc, bcr)
    return (logits_p[:N_NODES], emb_p[:N_NODES])
